# Initial kernel scaffold; baseline (speedup 1.0000x reference)
import functools

import jax
import jax.numpy as jnp
from jax import lax
from jax.experimental import pallas as pl
from jax.experimental.pallas import tpu as pltpu
from jax.experimental.pallas import tpu_sc as plsc

_CHUNK = 128
_NW = 32
_NTILE = 16


def _make_edge_pass(n_pad, d, n_ch):
    rpt = n_pad // _NTILE

    mesh = plsc.VectorSubcoreMesh(core_axis_name="c", subcore_axis_name="s")

    @functools.partial(
        pl.kernel,
        out_type=jax.ShapeDtypeStruct((2, n_pad, d), jnp.float32),
        mesh=mesh,
        scratch_types=[
            pltpu.VMEM((n_ch, _CHUNK), jnp.int32),
            pltpu.VMEM((n_ch, _CHUNK), jnp.int32),
            pltpu.VMEM((_CHUNK, d), jnp.float32),
            pltpu.VMEM_SHARED((n_pad, d), jnp.float32),
            pltpu.SemaphoreType.DMA,
        ],
    )
    def edge_pass(hn_hbm, srcw_hbm, dstw_hbm, zeros_hbm, out_hbm,
                  src_v, dst_v, rows_v, acc_sh, sem):
        c = lax.axis_index("c")
        s = lax.axis_index("s")
        wid = s * 2 + c
        pltpu.sync_copy(zeros_hbm.at[pl.ds(s * rpt, rpt)],
                        acc_sh.at[pl.ds(s * rpt, rpt)])
        pltpu.sync_copy(srcw_hbm.at[wid], src_v)
        pltpu.sync_copy(dstw_hbm.at[wid], dst_v)
        plsc.subcore_barrier()

        def body(j, carry):
            pltpu.async_copy(hn_hbm.at[src_v.at[j]], rows_v, sem).wait()
            pltpu.sync_copy(rows_v, acc_sh.at[dst_v.at[j]], add=True)
            return carry

        lax.fori_loop(0, n_ch, body, 0)
        plsc.subcore_barrier()
        pltpu.sync_copy(acc_sh.at[pl.ds(s * rpt, rpt)],
                        out_hbm.at[c, pl.ds(s * rpt, rpt)])

    return edge_pass


def _make_deg_pass(n_pad, n_ch):
    d = 16
    rpt = n_pad // _NTILE
    mesh = plsc.VectorSubcoreMesh(core_axis_name="c", subcore_axis_name="s")

    @functools.partial(
        pl.kernel,
        out_type=jax.ShapeDtypeStruct((2, n_pad, d), jnp.float32),
        mesh=mesh,
        scratch_types=[
            pltpu.VMEM((n_ch, _CHUNK), jnp.int32),
            pltpu.VMEM((_CHUNK, d), jnp.float32),
            pltpu.VMEM_SHARED((n_pad, d), jnp.float32),
        ],
    )
    def deg_pass(dstw_hbm, e1_hbm, zeros_hbm, out_hbm, dst_v, ones_v, acc_sh):
        c = lax.axis_index("c")
        s = lax.axis_index("s")
        wid = s * 2 + c
        pltpu.sync_copy(zeros_hbm.at[pl.ds(s * rpt, rpt)],
                        acc_sh.at[pl.ds(s * rpt, rpt)])
        pltpu.sync_copy(dstw_hbm.at[wid], dst_v)
        pltpu.sync_copy(e1_hbm, ones_v)
        plsc.subcore_barrier()

        def body(j, carry):
            pltpu.sync_copy(ones_v, acc_sh.at[dst_v.at[j]], add=True)
            return carry

        lax.fori_loop(0, n_ch, body, 0)
        plsc.subcore_barrier()
        pltpu.sync_copy(acc_sh.at[pl.ds(s * rpt, rpt)],
                        out_hbm.at[c, pl.ds(s * rpt, rpt)])

    return deg_pass


def _t1_body(n, br, x_ref, w1_ref, b1_ref, d0_ref, d1_ref, hn_ref, norm_ref):
    i = pl.program_id(0)
    deg = d0_ref[0, :, 0:1] + d1_ref[0, :, 0:1] + 1.0
    norm = lax.rsqrt(deg)
    row = lax.broadcasted_iota(jnp.int32, (br, 1), 0) + i * br
    norm = jnp.where(row < n, norm, 0.0)
    h = jnp.dot(x_ref[...], w1_ref[...], preferred_element_type=jnp.float32)
    h = h + b1_ref[...]
    hn_ref[...] = h * norm
    norm_ref[...] = norm


def _t3_body(p0_ref, p1_ref, hn_ref, norm_ref, w2_ref, b2_ref, hn2_ref):
    norm = norm_ref[...]
    agg = (p0_ref[0] + p1_ref[0] + hn_ref[...]) * norm
    r = jnp.maximum(agg, 0.0)
    h2 = jnp.dot(r, w2_ref[...], preferred_element_type=jnp.float32)
    h2 = h2 + b2_ref[...]
    hn2_ref[...] = h2 * norm


def _t4_body(n_cls, br, q0_ref, q1_ref, hn2_ref, norm_ref, out_ref):
    z = (q0_ref[0] + q1_ref[0] + hn2_ref[...]) * norm_ref[...]
    col = lax.broadcasted_iota(jnp.int32, (br, 16), 1)
    z = jnp.where(col < n_cls, z, -1e30)
    m = jnp.max(z, axis=-1, keepdims=True)
    e = jnp.exp(z - m)
    ssum = jnp.sum(e, axis=-1, keepdims=True)
    out_ref[...] = (z - m) - jnp.log(ssum)


def kernel(x, W1, b1, W2, b2, edge_index):
    n, d_in = x.shape
    d_hid = W1.shape[1]
    n_cls = W2.shape[1]
    e = edge_index.shape[1]
    f32 = jnp.float32

    n_ch = -(-e // (_NW * _CHUNK))
    e_pad = _NW * _CHUNK * n_ch
    n_pad = ((n + 1 + 127) // 128) * 128

    src = edge_index[0].astype(jnp.int32)
    dst = edge_index[1].astype(jnp.int32)
    pad = jnp.full((e_pad - e,), n, jnp.int32)
    srcw = jnp.concatenate([src, pad]).reshape(_NW, n_ch, _CHUNK)
    dstw = jnp.concatenate([dst, pad]).reshape(_NW, n_ch, _CHUNK)

    xp = jnp.pad(x, ((0, n_pad - n), (0, 0)))
    w2p = jnp.pad(W2, ((0, 0), (0, 16 - n_cls)))
    b1r = b1.reshape(1, d_hid)
    b2r = jnp.pad(b2, (0, 16 - n_cls)).reshape(1, 16)
    zeros_h = jnp.zeros((n_pad, d_hid), f32)
    zeros_c = jnp.zeros((n_pad, 16), f32)
    e1 = jnp.zeros((_CHUNK, 16), f32).at[:, 0].set(1.0)

    grid = 8
    br = n_pad // grid

    deg_parts = _make_deg_pass(n_pad, n_ch)(dstw, e1, zeros_c)

    hn, norm = pl.pallas_call(
        functools.partial(_t1_body, n, br),
        grid=(grid,),
        in_specs=[
            pl.BlockSpec((br, d_in), lambda i: (i, 0)),
            pl.BlockSpec((d_in, d_hid), lambda i: (0, 0)),
            pl.BlockSpec((1, d_hid), lambda i: (0, 0)),
            pl.BlockSpec((1, br, 16), lambda i: (0, i, 0)),
            pl.BlockSpec((1, br, 16), lambda i: (1, i, 0)),
        ],
        out_specs=[
            pl.BlockSpec((br, d_hid), lambda i: (i, 0)),
            pl.BlockSpec((br, 1), lambda i: (i, 0)),
        ],
        out_shape=[
            jax.ShapeDtypeStruct((n_pad, d_hid), f32),
            jax.ShapeDtypeStruct((n_pad, 1), f32),
        ],
    )(xp, W1, b1r, deg_parts, deg_parts)

    parts1 = _make_edge_pass(n_pad, d_hid, n_ch)(hn, srcw, dstw, zeros_h)

    hn2 = pl.pallas_call(
        _t3_body,
        grid=(grid,),
        in_specs=[
            pl.BlockSpec((1, br, d_hid), lambda i: (0, i, 0)),
            pl.BlockSpec((1, br, d_hid), lambda i: (1, i, 0)),
            pl.BlockSpec((br, d_hid), lambda i: (i, 0)),
            pl.BlockSpec((br, 1), lambda i: (i, 0)),
            pl.BlockSpec((d_hid, 16), lambda i: (0, 0)),
            pl.BlockSpec((1, 16), lambda i: (0, 0)),
        ],
        out_specs=pl.BlockSpec((br, 16), lambda i: (i, 0)),
        out_shape=jax.ShapeDtypeStruct((n_pad, 16), f32),
    )(parts1, parts1, hn, norm, w2p, b2r)

    parts2 = _make_edge_pass(n_pad, 16, n_ch)(hn2, srcw, dstw, zeros_c)

    out = pl.pallas_call(
        functools.partial(_t4_body, n_cls, br),
        grid=(grid,),
        in_specs=[
            pl.BlockSpec((1, br, 16), lambda i: (0, i, 0)),
            pl.BlockSpec((1, br, 16), lambda i: (1, i, 0)),
            pl.BlockSpec((br, 16), lambda i: (i, 0)),
            pl.BlockSpec((br, 1), lambda i: (i, 0)),
        ],
        out_specs=pl.BlockSpec((br, 16), lambda i: (i, 0)),
        out_shape=jax.ShapeDtypeStruct((n_pad, 16), f32),
    )(parts2, parts2, hn2, norm)

    return out[:n, :n_cls]


# speedup vs baseline: 9.0360x; 9.0360x over previous
import functools

import jax
import jax.numpy as jnp
from jax import lax
from jax.experimental import pallas as pl
from jax.experimental.pallas import tpu as pltpu
from jax.experimental.pallas import tpu_sc as plsc

_CHUNK = 128
_NW = 32
_NTILE = 16


def _make_edge_pass(n_pad, d, n_ch):
    rpt = n_pad // _NTILE

    mesh = plsc.VectorSubcoreMesh(core_axis_name="c", subcore_axis_name="s")

    @functools.partial(
        pl.kernel,
        out_type=jax.ShapeDtypeStruct((2, n_pad, d), jnp.float32),
        mesh=mesh,
        compiler_params=pltpu.CompilerParams(use_tc_tiling_on_sc=False),
        scratch_types=[
            pltpu.VMEM((n_ch, _CHUNK), jnp.int32),
            pltpu.VMEM((n_ch, _CHUNK), jnp.int32),
            pltpu.VMEM((_CHUNK, d), jnp.float32),
            pltpu.VMEM_SHARED((n_pad, d), jnp.float32),
            pltpu.SemaphoreType.DMA,
        ],
    )
    def edge_pass(hn_hbm, srcw_hbm, dstw_hbm, zeros_hbm, out_hbm,
                  src_v, dst_v, rows_v, acc_sh, sem):
        c = lax.axis_index("c")
        s = lax.axis_index("s")
        wid = s * 2 + c
        pltpu.sync_copy(zeros_hbm.at[pl.ds(s * rpt, rpt)],
                        acc_sh.at[pl.ds(s * rpt, rpt)])
        pltpu.sync_copy(srcw_hbm.at[wid], src_v)
        pltpu.sync_copy(dstw_hbm.at[wid], dst_v)
        plsc.subcore_barrier()

        def body(j, carry):
            pltpu.async_copy(hn_hbm.at[src_v.at[j]], rows_v, sem).wait()
            pltpu.sync_copy(rows_v, acc_sh.at[dst_v.at[j]], add=True)
            return carry

        lax.fori_loop(0, n_ch, body, 0)
        plsc.subcore_barrier()
        pltpu.sync_copy(acc_sh.at[pl.ds(s * rpt, rpt)],
                        out_hbm.at[c, pl.ds(s * rpt, rpt)])

    return edge_pass


def _make_deg_pass(n_pad, n_ch):
    d = 16
    rpt = n_pad // _NTILE
    mesh = plsc.VectorSubcoreMesh(core_axis_name="c", subcore_axis_name="s")

    @functools.partial(
        pl.kernel,
        out_type=jax.ShapeDtypeStruct((2, n_pad, d), jnp.float32),
        mesh=mesh,
        compiler_params=pltpu.CompilerParams(use_tc_tiling_on_sc=False),
        scratch_types=[
            pltpu.VMEM((n_ch, _CHUNK), jnp.int32),
            pltpu.VMEM((_CHUNK, d), jnp.float32),
            pltpu.VMEM_SHARED((n_pad, d), jnp.float32),
        ],
    )
    def deg_pass(dstw_hbm, e1_hbm, zeros_hbm, out_hbm, dst_v, ones_v, acc_sh):
        c = lax.axis_index("c")
        s = lax.axis_index("s")
        wid = s * 2 + c
        pltpu.sync_copy(zeros_hbm.at[pl.ds(s * rpt, rpt)],
                        acc_sh.at[pl.ds(s * rpt, rpt)])
        pltpu.sync_copy(dstw_hbm.at[wid], dst_v)
        pltpu.sync_copy(e1_hbm, ones_v)
        plsc.subcore_barrier()

        def body(j, carry):
            pltpu.sync_copy(ones_v, acc_sh.at[dst_v.at[j]], add=True)
            return carry

        lax.fori_loop(0, n_ch, body, 0)
        plsc.subcore_barrier()
        pltpu.sync_copy(acc_sh.at[pl.ds(s * rpt, rpt)],
                        out_hbm.at[c, pl.ds(s * rpt, rpt)])

    return deg_pass


def _t1_body(n, br, x_ref, w1_ref, b1_ref, d0_ref, d1_ref, hn_ref, norm_ref):
    i = pl.program_id(0)
    deg = d0_ref[0, :, 0:1] + d1_ref[0, :, 0:1] + 1.0
    norm = lax.rsqrt(deg)
    row = lax.broadcasted_iota(jnp.int32, (br, 1), 0) + i * br
    norm = jnp.where(row < n, norm, 0.0)
    h = jnp.dot(x_ref[...], w1_ref[...], preferred_element_type=jnp.float32)
    h = h + b1_ref[...]
    hn_ref[...] = h * norm
    norm_ref[...] = norm


def _t3_body(p0_ref, p1_ref, hn_ref, norm_ref, w2_ref, b2_ref, hn2_ref):
    norm = norm_ref[...]
    agg = (p0_ref[0] + p1_ref[0] + hn_ref[...]) * norm
    r = jnp.maximum(agg, 0.0)
    h2 = jnp.dot(r, w2_ref[...], preferred_element_type=jnp.float32)
    h2 = h2 + b2_ref[...]
    hn2_ref[...] = h2 * norm


def _t4_body(n_cls, br, q0_ref, q1_ref, hn2_ref, norm_ref, out_ref):
    z = (q0_ref[0] + q1_ref[0] + hn2_ref[...]) * norm_ref[...]
    col = lax.broadcasted_iota(jnp.int32, (br, 16), 1)
    z = jnp.where(col < n_cls, z, -1e30)
    m = jnp.max(z, axis=-1, keepdims=True)
    e = jnp.exp(z - m)
    ssum = jnp.sum(e, axis=-1, keepdims=True)
    out_ref[...] = (z - m) - jnp.log(ssum)


def kernel(x, W1, b1, W2, b2, edge_index):
    n, d_in = x.shape
    d_hid = W1.shape[1]
    n_cls = W2.shape[1]
    e = edge_index.shape[1]
    f32 = jnp.float32

    n_ch = -(-e // (_NW * _CHUNK))
    e_pad = _NW * _CHUNK * n_ch
    n_pad = ((n + 1 + 127) // 128) * 128

    src = edge_index[0].astype(jnp.int32)
    dst = edge_index[1].astype(jnp.int32)
    pad = jnp.full((e_pad - e,), n, jnp.int32)
    srcw = jnp.concatenate([src, pad]).reshape(_NW, n_ch, _CHUNK)
    dstw = jnp.concatenate([dst, pad]).reshape(_NW, n_ch, _CHUNK)

    xp = jnp.pad(x, ((0, n_pad - n), (0, 0)))
    w2p = jnp.pad(W2, ((0, 0), (0, 16 - n_cls)))
    b1r = b1.reshape(1, d_hid)
    b2r = jnp.pad(b2, (0, 16 - n_cls)).reshape(1, 16)
    zeros_h = jnp.zeros((n_pad, d_hid), f32)
    zeros_c = jnp.zeros((n_pad, 16), f32)
    e1 = jnp.zeros((_CHUNK, 16), f32).at[:, 0].set(1.0)

    grid = 8
    br = n_pad // grid

    deg_parts = _make_deg_pass(n_pad, n_ch)(dstw, e1, zeros_c)

    hn, norm = pl.pallas_call(
        functools.partial(_t1_body, n, br),
        grid=(grid,),
        in_specs=[
            pl.BlockSpec((br, d_in), lambda i: (i, 0)),
            pl.BlockSpec((d_in, d_hid), lambda i: (0, 0)),
            pl.BlockSpec((1, d_hid), lambda i: (0, 0)),
            pl.BlockSpec((1, br, 16), lambda i: (0, i, 0)),
            pl.BlockSpec((1, br, 16), lambda i: (1, i, 0)),
        ],
        out_specs=[
            pl.BlockSpec((br, d_hid), lambda i: (i, 0)),
            pl.BlockSpec((br, 1), lambda i: (i, 0)),
        ],
        out_shape=[
            jax.ShapeDtypeStruct((n_pad, d_hid), f32),
            jax.ShapeDtypeStruct((n_pad, 1), f32),
        ],
    )(xp, W1, b1r, deg_parts, deg_parts)

    parts1 = _make_edge_pass(n_pad, d_hid, n_ch)(hn, srcw, dstw, zeros_h)

    hn2 = pl.pallas_call(
        _t3_body,
        grid=(grid,),
        in_specs=[
            pl.BlockSpec((1, br, d_hid), lambda i: (0, i, 0)),
            pl.BlockSpec((1, br, d_hid), lambda i: (1, i, 0)),
            pl.BlockSpec((br, d_hid), lambda i: (i, 0)),
            pl.BlockSpec((br, 1), lambda i: (i, 0)),
            pl.BlockSpec((d_hid, 16), lambda i: (0, 0)),
            pl.BlockSpec((1, 16), lambda i: (0, 0)),
        ],
        out_specs=pl.BlockSpec((br, 16), lambda i: (i, 0)),
        out_shape=jax.ShapeDtypeStruct((n_pad, 16), f32),
    )(parts1, parts1, hn, norm, w2p, b2r)

    parts2 = _make_edge_pass(n_pad, 16, n_ch)(hn2, srcw, dstw, zeros_c)

    out = pl.pallas_call(
        functools.partial(_t4_body, n_cls, br),
        grid=(grid,),
        in_specs=[
            pl.BlockSpec((1, br, 16), lambda i: (0, i, 0)),
            pl.BlockSpec((1, br, 16), lambda i: (1, i, 0)),
            pl.BlockSpec((br, 16), lambda i: (i, 0)),
            pl.BlockSpec((br, 1), lambda i: (i, 0)),
        ],
        out_specs=pl.BlockSpec((br, 16), lambda i: (i, 0)),
        out_shape=jax.ShapeDtypeStruct((n_pad, 16), f32),
    )(parts2, parts2, hn2, norm)

    return out[:n, :n_cls]
